# baseline (speedup 1.0000x reference)
import functools

import jax
import jax.numpy as jnp
from jax import lax
from jax.experimental import pallas as pl
from jax.experimental.pallas import tpu as pltpu
from jax.experimental.pallas import tpu_sc as plsc

N_E = 8192
E_DIM = 256
BETA = 0.25

_M_TILE = 512


_DOT_N = 1024
_SCAN_N = 128


def _dist_argmin_body(z_ref, w_ref, zn_ref, wn_ref, idx_ref, minv_ref,
                      w2_ref):
    @pl.when(pl.program_id(0) == 0)
    def _():
        w2_ref[...] = w_ref[...] * -2.0

    zn = zn_ref[...]
    z = z_ref[...]
    m = jnp.full((_M_TILE, _SCAN_N), jnp.inf, jnp.float32)
    a = jnp.zeros((_M_TILE, _SCAN_N), jnp.float32)
    for g in range(N_E // _DOT_N):
        zw2 = lax.dot_general(
            z, w2_ref[g * _DOT_N:(g + 1) * _DOT_N, :],
            dimension_numbers=(((1,), (1,)), ((), ())),
            preferred_element_type=jnp.float32,
        )
        for c in range(_DOT_N // _SCAN_N):
            col0 = g * _DOT_N + c * _SCAN_N
            d = ((zn + wn_ref[:, col0:col0 + _SCAN_N])
                 + zw2[:, c * _SCAN_N:(c + 1) * _SCAN_N])
            cmp = d < m
            m = jnp.where(cmp, d, m)
            a = jnp.where(cmp, jnp.float32(col0), a)
    lmin = jnp.min(m, axis=1, keepdims=True)
    lanes = lax.broadcasted_iota(jnp.int32, (_M_TILE, _SCAN_N), 1)
    col = a + lanes.astype(jnp.float32)
    lidx = jnp.min(jnp.where(m == lmin, col, jnp.float32(2 * N_E)), axis=1,
                   keepdims=True)
    idx_ref[...] = lidx.astype(jnp.int32)
    minv_ref[...] = lmin


def _dist_argmin(z_flat, W, zn, wn):
    m = z_flat.shape[0]
    grid = (m // _M_TILE,)
    return pl.pallas_call(
        _dist_argmin_body,
        grid=grid,
        in_specs=[
            pl.BlockSpec((_M_TILE, E_DIM), lambda i: (i, 0)),
            pl.BlockSpec((N_E, E_DIM), lambda i: (0, 0)),
            pl.BlockSpec((_M_TILE, 1), lambda i: (i, 0)),
            pl.BlockSpec((1, N_E), lambda i: (0, 0)),
        ],
        out_specs=[
            pl.BlockSpec((_M_TILE, 1), lambda i: (i, 0)),
            pl.BlockSpec((_M_TILE, 1), lambda i: (i, 0)),
        ],
        out_shape=[
            jax.ShapeDtypeStruct((m, 1), jnp.int32),
            jax.ShapeDtypeStruct((m, 1), jnp.float32),
        ],
        scratch_shapes=[pltpu.VMEM((N_E, E_DIM), jnp.float32)],
    )(z_flat, W, zn, wn)


def _make_sc_gather(n_rows):
    info = plsc.get_sparse_core_info()
    nw = info.num_cores * info.num_subcores
    rows_per_w = n_rows // nw
    n_chunks = max(1, rows_per_w // 128)
    chunk = rows_per_w // n_chunks
    mesh = plsc.VectorSubcoreMesh(core_axis_name="c", subcore_axis_name="s")

    @functools.partial(
        pl.kernel,
        mesh=mesh,
        out_type=jax.ShapeDtypeStruct((n_rows, E_DIM), jnp.float32),
        scratch_types=[
            pltpu.VMEM((n_chunks, chunk), jnp.int32),
            pltpu.VMEM((rows_per_w, E_DIM), jnp.float32),
            pltpu.SemaphoreType.DMA,
        ],
    )
    def gather_k(w_hbm, idx_hbm, out_hbm, idx_v, rows_v, sem):
        wid = lax.axis_index("s") * info.num_cores + lax.axis_index("c")
        base = wid * rows_per_w
        pltpu.sync_copy(idx_hbm.at[wid], idx_v)
        copies = [
            pltpu.async_copy(w_hbm.at[idx_v.at[j]],
                             rows_v.at[pl.ds(j * chunk, chunk)], sem)
            for j in range(n_chunks)
        ]
        for c in copies:
            c.wait()
        pltpu.sync_copy(rows_v, out_hbm.at[pl.ds(base, rows_per_w)])

    def run(w, idx):
        return gather_k(w, idx.reshape(nw, n_chunks, chunk))

    return run


def kernel(z, W):
    zp = jnp.transpose(z, (0, 2, 3, 1))
    z_flat = zp.reshape(-1, E_DIM)
    zn = jnp.sum(z_flat ** 2, axis=1, keepdims=True)
    wn = jnp.sum(W ** 2, axis=1)

    idx2d, minv = _dist_argmin(z_flat, W, zn, wn.reshape(1, N_E))
    idx = idx2d.reshape(-1)

    z_q = _make_sc_gather(z_flat.shape[0])(W, idx).reshape(zp.shape)

    n_elems = z_flat.shape[0] * E_DIM
    mean_sq = jnp.sum(minv) / n_elems
    loss = mean_sq + BETA * mean_sq

    z_q = zp + (z_q - zp)
    z_q = jnp.transpose(z_q, (0, 3, 1, 2))
    return (z_q, loss, (None, None, idx))


# speedup vs baseline: 1.0207x; 1.0207x over previous
import functools

import jax
import jax.numpy as jnp
from jax import lax
from jax.experimental import pallas as pl
from jax.experimental.pallas import tpu as pltpu
from jax.experimental.pallas import tpu_sc as plsc

N_E = 8192
E_DIM = 256
BETA = 0.25

_M_TILE = 512


_DOT_N = 1024
_SCAN_N = 128


def _dist_argmin_body(z_ref, w_ref, zn_ref, wn_ref, idx_ref, minv_ref,
                      w2_ref):
    @pl.when(pl.program_id(0) == 0)
    def _():
        w2_ref[...] = (w_ref[...] * -2.0).astype(jnp.bfloat16)

    zn = zn_ref[...]
    z = z_ref[...]
    m = jnp.full((_M_TILE, _SCAN_N), jnp.inf, jnp.float32)
    a = jnp.zeros((_M_TILE, _SCAN_N), jnp.float32)
    for g in range(N_E // _DOT_N):
        zw2 = lax.dot_general(
            z, w2_ref[g * _DOT_N:(g + 1) * _DOT_N, :],
            dimension_numbers=(((1,), (1,)), ((), ())),
            preferred_element_type=jnp.float32,
        )
        for c in range(_DOT_N // _SCAN_N):
            col0 = g * _DOT_N + c * _SCAN_N
            d = ((zn + wn_ref[:, col0:col0 + _SCAN_N])
                 + zw2[:, c * _SCAN_N:(c + 1) * _SCAN_N])
            cmp = d < m
            m = jnp.where(cmp, d, m)
            a = jnp.where(cmp, jnp.float32(col0), a)
    lmin = jnp.min(m, axis=1, keepdims=True)
    lanes = lax.broadcasted_iota(jnp.int32, (_M_TILE, _SCAN_N), 1)
    col = a + lanes.astype(jnp.float32)
    lidx = jnp.min(jnp.where(m == lmin, col, jnp.float32(2 * N_E)), axis=1,
                   keepdims=True)
    idx_ref[...] = lidx.astype(jnp.int32)
    minv_ref[...] = lmin


def _dist_argmin(z_flat, W, zn, wn):
    m = z_flat.shape[0]
    grid = (m // _M_TILE,)
    return pl.pallas_call(
        _dist_argmin_body,
        grid=grid,
        in_specs=[
            pl.BlockSpec((_M_TILE, E_DIM), lambda i: (i, 0)),
            pl.BlockSpec((N_E, E_DIM), lambda i: (0, 0)),
            pl.BlockSpec((_M_TILE, 1), lambda i: (i, 0)),
            pl.BlockSpec((1, N_E), lambda i: (0, 0)),
        ],
        out_specs=[
            pl.BlockSpec((_M_TILE, 1), lambda i: (i, 0)),
            pl.BlockSpec((_M_TILE, 1), lambda i: (i, 0)),
        ],
        out_shape=[
            jax.ShapeDtypeStruct((m, 1), jnp.int32),
            jax.ShapeDtypeStruct((m, 1), jnp.float32),
        ],
        scratch_shapes=[pltpu.VMEM((N_E, E_DIM), jnp.bfloat16)],
    )(z_flat, W, zn, wn)


def _make_sc_gather(n_rows):
    info = plsc.get_sparse_core_info()
    nw = info.num_cores * info.num_subcores
    rows_per_w = n_rows // nw
    n_chunks = max(1, rows_per_w // 128)
    chunk = rows_per_w // n_chunks
    mesh = plsc.VectorSubcoreMesh(core_axis_name="c", subcore_axis_name="s")

    @functools.partial(
        pl.kernel,
        mesh=mesh,
        out_type=jax.ShapeDtypeStruct((n_rows, E_DIM), jnp.float32),
        scratch_types=[
            pltpu.VMEM((n_chunks, chunk), jnp.int32),
            pltpu.VMEM((rows_per_w, E_DIM), jnp.float32),
            pltpu.SemaphoreType.DMA,
        ],
    )
    def gather_k(w_hbm, idx_hbm, out_hbm, idx_v, rows_v, sem):
        wid = lax.axis_index("s") * info.num_cores + lax.axis_index("c")
        base = wid * rows_per_w
        pltpu.sync_copy(idx_hbm.at[wid], idx_v)
        copies = [
            pltpu.async_copy(w_hbm.at[idx_v.at[j]],
                             rows_v.at[pl.ds(j * chunk, chunk)], sem)
            for j in range(n_chunks)
        ]
        for c in copies:
            c.wait()
        pltpu.sync_copy(rows_v, out_hbm.at[pl.ds(base, rows_per_w)])

    def run(w, idx):
        return gather_k(w, idx.reshape(nw, n_chunks, chunk))

    return run


def kernel(z, W):
    zp = jnp.transpose(z, (0, 2, 3, 1))
    z_flat = zp.reshape(-1, E_DIM)
    zn = jnp.sum(z_flat ** 2, axis=1, keepdims=True)
    wn = jnp.sum(W ** 2, axis=1)

    idx2d, minv = _dist_argmin(z_flat, W, zn, wn.reshape(1, N_E))
    idx = idx2d.reshape(-1)

    z_q = _make_sc_gather(z_flat.shape[0])(W, idx).reshape(zp.shape)

    n_elems = z_flat.shape[0] * E_DIM
    mean_sq = jnp.sum(minv) / n_elems
    loss = mean_sq + BETA * mean_sq

    z_q = zp + (z_q - zp)
    z_q = jnp.transpose(z_q, (0, 3, 1, 2))
    return (z_q, loss, (None, None, idx))


# speedup vs baseline: 1.2588x; 1.2333x over previous
import functools

import jax
import jax.numpy as jnp
from jax import lax
from jax.experimental import pallas as pl
from jax.experimental.pallas import tpu as pltpu
from jax.experimental.pallas import tpu_sc as plsc

N_E = 8192
E_DIM = 256
BETA = 0.25

_M_TILE = 2048


_DOT_N = 512
_SCAN_N = 128


def _dist_argmin_body(z_ref, w_ref, wn_ref, idx_ref, minv_ref, w2_ref):
    @pl.when(pl.program_id(0) == 0)
    def _():
        w2_ref[...] = (w_ref[...] * -2.0).astype(jnp.bfloat16)

    z = z_ref[...]
    zn = jnp.sum(z * z, axis=1, keepdims=True)
    m = jnp.full((_M_TILE, _SCAN_N), jnp.inf, jnp.float32)
    a = jnp.zeros((_M_TILE, _SCAN_N), jnp.float32)
    for g in range(N_E // _DOT_N):
        zw2 = lax.dot_general(
            z, w2_ref[g * _DOT_N:(g + 1) * _DOT_N, :],
            dimension_numbers=(((1,), (1,)), ((), ())),
            preferred_element_type=jnp.float32,
        )
        for c in range(_DOT_N // _SCAN_N):
            col0 = g * _DOT_N + c * _SCAN_N
            d = ((zn + wn_ref[:, col0:col0 + _SCAN_N])
                 + zw2[:, c * _SCAN_N:(c + 1) * _SCAN_N])
            cmp = d < m
            m = jnp.where(cmp, d, m)
            a = jnp.where(cmp, jnp.float32(col0), a)
    lmin = jnp.min(m, axis=1, keepdims=True)
    lanes = lax.broadcasted_iota(jnp.int32, (_M_TILE, _SCAN_N), 1)
    col = a + lanes.astype(jnp.float32)
    lidx = jnp.min(jnp.where(m == lmin, col, jnp.float32(2 * N_E)), axis=1,
                   keepdims=True)
    idx_ref[...] = lidx.astype(jnp.int32)
    minv_ref[...] = lmin


def _dist_argmin(z_flat, W, wn):
    m = z_flat.shape[0]
    grid = (m // _M_TILE,)
    return pl.pallas_call(
        _dist_argmin_body,
        grid=grid,
        in_specs=[
            pl.BlockSpec((_M_TILE, E_DIM), lambda i: (i, 0)),
            pl.BlockSpec((N_E, E_DIM), lambda i: (0, 0)),
            pl.BlockSpec((1, N_E), lambda i: (0, 0)),
        ],
        out_specs=[
            pl.BlockSpec((_M_TILE, 1), lambda i: (i, 0)),
            pl.BlockSpec((_M_TILE, 1), lambda i: (i, 0)),
        ],
        out_shape=[
            jax.ShapeDtypeStruct((m, 1), jnp.int32),
            jax.ShapeDtypeStruct((m, 1), jnp.float32),
        ],
        scratch_shapes=[pltpu.VMEM((N_E, E_DIM), jnp.bfloat16)],
    )(z_flat, W, wn)


def _make_sc_gather(n_rows):
    info = plsc.get_sparse_core_info()
    nw = info.num_cores * info.num_subcores
    rows_per_w = n_rows // nw
    n_chunks = max(1, rows_per_w // 128)
    chunk = rows_per_w // n_chunks
    mesh = plsc.VectorSubcoreMesh(core_axis_name="c", subcore_axis_name="s")

    @functools.partial(
        pl.kernel,
        mesh=mesh,
        out_type=jax.ShapeDtypeStruct((n_rows, E_DIM), jnp.float32),
        scratch_types=[
            pltpu.VMEM((n_chunks, chunk), jnp.int32),
            pltpu.VMEM((rows_per_w, E_DIM), jnp.float32),
            pltpu.SemaphoreType.DMA,
            pltpu.SemaphoreType.DMA,
        ],
    )
    def gather_k(w_hbm, idx_hbm, out_hbm, idx_v, rows_v, sem, sem_out):
        wid = lax.axis_index("s") * info.num_cores + lax.axis_index("c")
        base = wid * rows_per_w
        pltpu.sync_copy(idx_hbm.at[wid], idx_v)
        copies = [
            pltpu.async_copy(w_hbm.at[idx_v.at[j]],
                             rows_v.at[pl.ds(j * chunk, chunk)], sem)
            for j in range(n_chunks)
        ]
        out_copies = []
        for j, c in enumerate(copies):
            c.wait()
            out_copies.append(pltpu.async_copy(
                rows_v.at[pl.ds(j * chunk, chunk)],
                out_hbm.at[pl.ds(base + j * chunk, chunk)], sem_out))
        for c in out_copies:
            c.wait()

    def run(w, idx):
        return gather_k(w, idx.reshape(nw, n_chunks, chunk))

    return run


def kernel(z, W):
    zp = jnp.transpose(z, (0, 2, 3, 1))
    z_flat = zp.reshape(-1, E_DIM)
    wn = jnp.sum(W ** 2, axis=1)

    idx2d, minv = _dist_argmin(z_flat, W, wn.reshape(1, N_E))
    idx = idx2d.reshape(-1)

    z_q = _make_sc_gather(z_flat.shape[0])(W, idx).reshape(zp.shape)

    n_elems = z_flat.shape[0] * E_DIM
    mean_sq = jnp.sum(minv) / n_elems
    loss = mean_sq + BETA * mean_sq

    z_q = jnp.transpose(z_q, (0, 3, 1, 2))
    return (z_q, loss, (None, None, idx))
